# baseline (speedup 1.0000x reference)
import functools

import jax
import jax.numpy as jnp
from jax import lax
from jax.experimental import pallas as pl
from jax.experimental.pallas import tpu as pltpu
from jax.experimental.pallas import tpu_sc as plsc

N = 10000
E = 320000
FT = 128
HID = 128
NS = 16
CK = 128
NCHT = E // CK
CPS = NCHT // NS
XTRA = NCHT - CPS * NS
DR = 3
DI = 6
UN = 6
TRIP = (CPS + 1 + 2 + UN - 1) // UN
RPS = 624
TAIL = N - NS * RPS


def _mm_body(s1_ref, s2_ref, w_ref, o_ref):
    o_ref[0] = jnp.dot(s1_ref[0], w_ref[...], preferred_element_type=jnp.float32)
    o_ref[1] = jnp.dot(s2_ref[0], w_ref[...], preferred_element_type=jnp.float32)


def _mm(seq1, seq2, W):
    BR = 2000
    return pl.pallas_call(
        _mm_body,
        grid=(N // BR,),
        in_specs=[pl.BlockSpec((1, BR, FT), lambda i: (0, i, 0)),
                  pl.BlockSpec((1, BR, FT), lambda i: (0, i, 0)),
                  pl.BlockSpec((FT, HID), lambda i: (0, 0))],
        out_specs=pl.BlockSpec((2, BR, HID), lambda i: (0, i, 0)),
        out_shape=jax.ShapeDtypeStruct((2, N, HID), jnp.float32),
    )(seq1, seq2, W)


def _sc_agg(fts2, adj2, zrows):
    @functools.partial(
        pl.kernel,
        out_type=jax.ShapeDtypeStruct((2 * N, HID), jnp.float32),
        mesh=plsc.VectorSubcoreMesh(core_axis_name="c", subcore_axis_name="s"),
        scratch_types=(
            [pltpu.VMEM((CK,), jnp.int32) for _ in range(DI)]
            + [pltpu.VMEM((CK,), jnp.int32) for _ in range(DI)]
            + [pltpu.VMEM((CK, HID), jnp.float32) for _ in range(DR)]
            + [pltpu.SemaphoreType.DMA] * (DI + 2 * DR)
            + [pltpu.VMEM_SHARED((N, HID), jnp.float32)]
        ),
    )
    def k(fts_h, adj_h, z_h, out_h, *refs):
        src_i = refs[0:DI]
        dst_i = refs[DI:2 * DI]
        rows = refs[2 * DI:2 * DI + DR]
        sem_i = refs[2 * DI + DR:3 * DI + DR]
        sem_g = refs[3 * DI + DR:3 * DI + 2 * DR]
        sem_s = refs[3 * DI + 2 * DR:3 * DI + 3 * DR]
        acc = refs[-1]
        cid = lax.axis_index("c")
        sid = lax.axis_index("s")
        ftsg = fts_h.at[pl.ds(cid * N, N)]

        pltpu.sync_copy(z_h, acc.at[pl.ds(sid * RPS, RPS)])

        @pl.when(sid == 0)
        def _():
            pltpu.sync_copy(z_h.at[pl.ds(0, TAIL)],
                            acc.at[pl.ds(NS * RPS, TAIL)])

        plsc.subcore_barrier()

        ne = CPS + jnp.where(sid < XTRA, 1, 0)

        def chunk_id(j):
            return jnp.where(j < CPS, sid * CPS + j, NCHT - XTRA + sid)

        def idx_load(j, u):
            q = chunk_id(j)
            pltpu.async_copy(adj_h.at[pl.ds(q * CK, CK)], src_i[u], sem_i[u])
            pltpu.async_copy(adj_h.at[pl.ds(E + q * CK, CK)],
                             dst_i[u], sem_i[u])

        def idx_wait(u):
            pltpu.make_async_copy(adj_h.at[pl.ds(0, CK)],
                                  src_i[u], sem_i[u]).wait()
            pltpu.make_async_copy(adj_h.at[pl.ds(0, CK)],
                                  dst_i[u], sem_i[u]).wait()

        def g_issue(u, p):
            pltpu.async_copy(ftsg.at[src_i[u]], rows[p], sem_g[p])

        def g_wait(p):
            pltpu.make_async_copy(fts_h.at[pl.ds(0, CK)],
                                  rows[p], sem_g[p]).wait()

        def s_issue(p, u):
            pltpu.async_copy(rows[p], acc.at[dst_i[u]], sem_s[p], add=True)

        def s_wait(p):
            pltpu.make_async_copy(fts_h.at[pl.ds(0, CK)],
                                  rows[p], sem_s[p]).wait()

        for q in range(4):
            idx_load(q, q)
        for q in range(2):
            idx_wait(q)
            g_issue(q, q)

        def body(t, carry):
            j0 = t * UN
            for kk in range(UN):
                j = j0 + kk

                @pl.when(j < ne)
                def _(p=kk % DR, u=kk % DI):
                    g_wait(p)
                    s_issue(p, u)

                @pl.when((j >= 1) & (j < ne + 1))
                def _(p=(kk + 2) % DR):
                    s_wait(p)

                @pl.when(j < ne - 2)
                def _(p=(kk + 2) % DR, u=(kk + 2) % DI):
                    idx_wait(u)
                    g_issue(u, p)

                @pl.when(j < ne - 4)
                def _(u=(kk + 4) % DI):
                    idx_load(j + 4, u)

            return carry

        lax.fori_loop(0, TRIP, body, 0)
        plsc.subcore_barrier()
        pltpu.sync_copy(acc.at[pl.ds(sid * RPS, RPS)],
                        out_h.at[pl.ds(cid * N + sid * RPS, RPS)])

        @pl.when(sid == 0)
        def _():
            pltpu.sync_copy(acc.at[pl.ds(NS * RPS, TAIL)],
                            out_h.at[pl.ds(cid * N + NS * RPS, TAIL)])

    return k(fts2, adj2, zrows)


def _post_body(agg_ref, b_ref, a_ref, m_ref, sb_ref, h_ref, ret_ref, cm_ref):
    g = pl.program_id(0)
    x = agg_ref[0] + b_ref[...]
    a = a_ref[0, 0]
    h = jnp.where(x > 0.0, x, a * x)

    @pl.when(g == 0)
    def _():
        h_ref[0] = h
        c = jax.nn.sigmoid(jnp.sum(h, axis=0, keepdims=True) * (1.0 / N))
        cm_ref[...] = jnp.dot(c, m_ref[...], preferred_element_type=jnp.float32)

    s = jnp.sum(h * cm_ref[...], axis=-1)
    ret_ref[...] = s.reshape(1, 1, N) + sb_ref[...]


def _post(agg, b2, a2, M, sb):
    return pl.pallas_call(
        _post_body,
        grid=(2,),
        in_specs=[pl.BlockSpec((1, N, HID), lambda g: (g, 0, 0)),
                  pl.BlockSpec((1, HID), lambda g: (0, 0)),
                  pl.BlockSpec((1, 1), lambda g: (0, 0)),
                  pl.BlockSpec((HID, HID), lambda g: (0, 0)),
                  pl.BlockSpec((1, 1, N), lambda g: (g, 0, 0))],
        out_specs=[pl.BlockSpec((1, N, HID), lambda g: (0, 0, 0)),
                   pl.BlockSpec((1, 1, N), lambda g: (g, 0, 0))],
        out_shape=[jax.ShapeDtypeStruct((1, N, HID), jnp.float32),
                   jax.ShapeDtypeStruct((2, 1, N), jnp.float32)],
        scratch_shapes=[pltpu.VMEM((1, HID), jnp.float32)],
    )(agg, b2, a2, M, sb)


def kernel(seq1, seq2, adj, sparse, msk, samp_bias1, samp_bias2, W, b_gcn,
           prelu_a, M, disc_bias):
    fts2 = _mm(seq1, seq2, W).reshape(2 * N, HID)
    zrows = jnp.zeros((RPS, HID), jnp.float32)
    agg = _sc_agg(fts2, adj.reshape(2 * E), zrows).reshape(2, N, HID)
    b2 = b_gcn.reshape(1, HID)
    a2 = prelu_a.reshape(1, 1)
    sb = (jnp.stack([samp_bias1[0], samp_bias2[0]])
          .reshape(2, 1, N) + disc_bias)
    h1, ret2 = _post(agg, b2, a2, M, sb)
    ret = ret2.reshape(1, 2 * N)
    return (ret, h1)


# speedup vs baseline: 1.0156x; 1.0156x over previous
import functools

import jax
import jax.numpy as jnp
from jax import lax
from jax.experimental import pallas as pl
from jax.experimental.pallas import tpu as pltpu
from jax.experimental.pallas import tpu_sc as plsc

N = 10000
E = 320000
FT = 128
HID = 128
NS = 16
CK = 128
NCHT = E // CK
CPS = NCHT // NS
XTRA = NCHT - CPS * NS
DR = 3
DI = 6
UN = 6
TRIP = (CPS + 1 + 2 + UN - 1) // UN
RPS = 624
TAIL = N - NS * RPS


def _mm_body(s1_ref, s2_ref, w_ref, o_ref):
    o_ref[0] = jnp.dot(s1_ref[0], w_ref[...], preferred_element_type=jnp.float32)
    o_ref[1] = jnp.dot(s2_ref[0], w_ref[...], preferred_element_type=jnp.float32)


def _mm(seq1, seq2, W):
    BR = 2000
    return pl.pallas_call(
        _mm_body,
        grid=(N // BR,),
        in_specs=[pl.BlockSpec((1, BR, FT), lambda i: (0, i, 0)),
                  pl.BlockSpec((1, BR, FT), lambda i: (0, i, 0)),
                  pl.BlockSpec((FT, HID), lambda i: (0, 0))],
        out_specs=pl.BlockSpec((2, BR, HID), lambda i: (0, i, 0)),
        out_shape=jax.ShapeDtypeStruct((2, N, HID), jnp.float32),
    )(seq1, seq2, W)


def _sc_agg(fts2, adj2, zrows):
    @functools.partial(
        pl.kernel,
        out_type=jax.ShapeDtypeStruct((2 * N, HID), jnp.float32),
        mesh=plsc.VectorSubcoreMesh(core_axis_name="c", subcore_axis_name="s"),
        scratch_types=(
            [pltpu.VMEM((CK,), jnp.int32) for _ in range(DI)]
            + [pltpu.VMEM((CK,), jnp.int32) for _ in range(DI)]
            + [pltpu.VMEM((CK, HID), jnp.float32) for _ in range(DR)]
            + [pltpu.SemaphoreType.DMA] * (DI + 2 * DR)
            + [pltpu.VMEM_SHARED((N, HID), jnp.float32)]
        ),
    )
    def k(fts_h, adj_h, z_h, out_h, *refs):
        src_i = refs[0:DI]
        dst_i = refs[DI:2 * DI]
        rows = refs[2 * DI:2 * DI + DR]
        sem_i = refs[2 * DI + DR:3 * DI + DR]
        sem_g = refs[3 * DI + DR:3 * DI + 2 * DR]
        sem_s = refs[3 * DI + 2 * DR:3 * DI + 3 * DR]
        acc = refs[-1]
        cid = lax.axis_index("c")
        sid = lax.axis_index("s")
        ftsg = fts_h.at[pl.ds(cid * N, N)]

        pltpu.sync_copy(z_h, acc.at[pl.ds(sid * RPS, RPS)])

        @pl.when(sid == 0)
        def _():
            pltpu.sync_copy(z_h.at[pl.ds(0, TAIL)],
                            acc.at[pl.ds(NS * RPS, TAIL)])

        plsc.subcore_barrier()

        ne = CPS + jnp.where(sid < XTRA, 1, 0)

        def chunk_id(j):
            return jnp.where(j < CPS, sid * CPS + j, NCHT - XTRA + sid)

        def idx_load(j, u):
            q = chunk_id(j)
            pltpu.async_copy(adj_h.at[pl.ds(q * CK, CK)], src_i[u], sem_i[u])
            pltpu.async_copy(adj_h.at[pl.ds(E + q * CK, CK)],
                             dst_i[u], sem_i[u])

        def idx_wait(u):
            pltpu.make_async_copy(adj_h.at[pl.ds(0, CK)],
                                  src_i[u], sem_i[u]).wait()
            pltpu.make_async_copy(adj_h.at[pl.ds(0, CK)],
                                  dst_i[u], sem_i[u]).wait()

        def g_issue(u, p):
            pltpu.async_copy(ftsg.at[src_i[u]], rows[p], sem_g[p])

        def g_wait(p):
            pltpu.make_async_copy(fts_h.at[pl.ds(0, CK)],
                                  rows[p], sem_g[p]).wait()

        def s_issue(p, u):
            pltpu.async_copy(rows[p], acc.at[dst_i[u]], sem_s[p], add=True)

        def s_wait(p):
            pltpu.make_async_copy(fts_h.at[pl.ds(0, CK)],
                                  rows[p], sem_s[p]).wait()

        for q in range(4):
            idx_load(q, q)
        for q in range(2):
            idx_wait(q)
            g_issue(q, q)

        def body(t, carry):
            j0 = t * UN
            for kk in range(UN):
                j = j0 + kk

                @pl.when(j < ne)
                def _(p=kk % DR, u=kk % DI):
                    g_wait(p)
                    s_issue(p, u)

                @pl.when((j >= 1) & (j < ne + 1))
                def _(p=(kk + 2) % DR):
                    s_wait(p)

                @pl.when(j < ne - 2)
                def _(p=(kk + 2) % DR, u=(kk + 2) % DI):
                    idx_wait(u)
                    g_issue(u, p)

                @pl.when(j < ne - 4)
                def _(u=(kk + 4) % DI):
                    idx_load(j + 4, u)

            return carry

        lax.fori_loop(0, TRIP, body, 0)
        plsc.subcore_barrier()
        pltpu.sync_copy(acc.at[pl.ds(sid * RPS, RPS)],
                        out_h.at[pl.ds(cid * N + sid * RPS, RPS)])

        @pl.when(sid == 0)
        def _():
            pltpu.sync_copy(acc.at[pl.ds(NS * RPS, TAIL)],
                            out_h.at[pl.ds(cid * N + NS * RPS, TAIL)])

    return k(fts2, adj2, zrows)


def _post_body(agg_ref, b_ref, a_ref, m_ref, sb_ref, h_ref, ret_ref, cm_ref):
    g = pl.program_id(0)
    x = agg_ref[0] + b_ref[...]
    a = a_ref[0, 0]
    h = jnp.where(x > 0.0, x, a * x)

    @pl.when(g == 0)
    def _():
        h_ref[0] = h
        c = jax.nn.sigmoid(jnp.sum(h, axis=0, keepdims=True) * (1.0 / N))
        cm_ref[...] = jnp.dot(c, m_ref[...], preferred_element_type=jnp.float32)

    s = lax.dot_general(cm_ref[...], h, (((1,), (1,)), ((), ())),
                        precision=lax.Precision.HIGHEST,
                        preferred_element_type=jnp.float32)
    ret_ref[...] = s[None] + sb_ref[...]


def _post(agg, b2, a2, M, sb):
    return pl.pallas_call(
        _post_body,
        grid=(2,),
        in_specs=[pl.BlockSpec((1, N, HID), lambda g: (g, 0, 0)),
                  pl.BlockSpec((1, HID), lambda g: (0, 0)),
                  pl.BlockSpec((1, 1), lambda g: (0, 0)),
                  pl.BlockSpec((HID, HID), lambda g: (0, 0)),
                  pl.BlockSpec((1, 1, N), lambda g: (g, 0, 0))],
        out_specs=[pl.BlockSpec((1, N, HID), lambda g: (0, 0, 0)),
                   pl.BlockSpec((1, 1, N), lambda g: (g, 0, 0))],
        out_shape=[jax.ShapeDtypeStruct((1, N, HID), jnp.float32),
                   jax.ShapeDtypeStruct((2, 1, N), jnp.float32)],
        scratch_shapes=[pltpu.VMEM((1, HID), jnp.float32)],
    )(agg, b2, a2, M, sb)


def kernel(seq1, seq2, adj, sparse, msk, samp_bias1, samp_bias2, W, b_gcn,
           prelu_a, M, disc_bias):
    fts2 = _mm(seq1, seq2, W).reshape(2 * N, HID)
    zrows = jnp.zeros((RPS, HID), jnp.float32)
    agg = _sc_agg(fts2, adj.reshape(2 * E), zrows).reshape(2, N, HID)
    b2 = b_gcn.reshape(1, HID)
    a2 = prelu_a.reshape(1, 1)
    sb = (jnp.stack([samp_bias1[0], samp_bias2[0]])
          .reshape(2, 1, N) + disc_bias)
    h1, ret2 = _post(agg, b2, a2, M, sb)
    ret = ret2.reshape(1, 2 * N)
    return (ret, h1)
